# baseline (speedup 1.0000x reference)
import functools

import jax
import jax.numpy as jnp
from jax import lax
from jax.experimental import pallas as pl
from jax.experimental.pallas import tpu as pltpu
from jax.experimental.pallas import tpu_sc as plsc

N_NODES = 10000
N_EDGES = 160000
D_FEAT = 256
D_EDGE = 16
D_HID = 1024
D_OUT = 256
D_PACK = D_FEAT // 2

NUM_CORES = 2
NUM_SUBCORES = 16
N_WORKERS = NUM_CORES * NUM_SUBCORES
N_SLICES = 5
E_SLICE = N_EDGES // N_SLICES
EDGES_PER_WORKER = E_SLICE // N_WORKERS
CHUNK = 40
N_CHUNKS = EDGES_PER_WORKER // CHUNK


@functools.lru_cache(maxsize=None)
def _make_sc_gather():
    mesh = plsc.VectorSubcoreMesh(
        core_axis_name="c", subcore_axis_name="s",
        num_cores=NUM_CORES, num_subcores=NUM_SUBCORES)

    @functools.partial(
        pl.kernel,
        out_type=(
            jax.ShapeDtypeStruct((E_SLICE, D_PACK), jnp.int32),
            jax.ShapeDtypeStruct((E_SLICE, D_PACK), jnp.int32),
        ),
        mesh=mesh,
        scratch_types=[
            pltpu.VMEM((2, CHUNK), jnp.int32),
            pltpu.VMEM((2, CHUNK), jnp.int32),
            pltpu.VMEM((2, CHUNK, D_PACK), jnp.int32),
            pltpu.VMEM((2, CHUNK, D_PACK), jnp.int32),
        ] + [pltpu.SemaphoreType.DMA] * 12,
    )
    def _sc_gather(nodes_hbm, senders_hbm, receivers_hbm,
                   send_out, recv_out, sidx_v, ridx_v, srows_v, rrows_v,
                   *sems):
        (si_sem0, si_sem1, ri_sem0, ri_sem1, sg_sem0, sg_sem1,
         rg_sem0, rg_sem1, sw_sem0, sw_sem1, rw_sem0, rw_sem1) = sems
        si_sems = (si_sem0, si_sem1)
        ri_sems = (ri_sem0, ri_sem1)
        sg_sems = (sg_sem0, sg_sem1)
        rg_sems = (rg_sem0, rg_sem1)
        sw_sems = (sw_sem0, sw_sem1)
        rw_sems = (rw_sem0, rw_sem1)

        wid = lax.axis_index("s") * NUM_CORES + lax.axis_index("c")
        base = wid * EDGES_PER_WORKER

        def idx_load(i):
            b = i % 2
            off = base + i * CHUNK
            return (
                pltpu.async_copy(senders_hbm.at[pl.ds(off, CHUNK)],
                                 sidx_v.at[b], si_sems[b]),
                pltpu.async_copy(receivers_hbm.at[pl.ds(off, CHUNK)],
                                 ridx_v.at[b], ri_sems[b]),
            )

        def gather(i):
            b = i % 2
            return (
                pltpu.async_copy(nodes_hbm.at[sidx_v.at[b]], srows_v.at[b],
                                 sg_sems[b]),
                pltpu.async_copy(nodes_hbm.at[ridx_v.at[b]], rrows_v.at[b],
                                 rg_sems[b]),
            )

        def writeback(i):
            b = i % 2
            off = base + i * CHUNK
            return (
                pltpu.async_copy(srows_v.at[b], send_out.at[pl.ds(off, CHUNK)],
                                 sw_sems[b]),
                pltpu.async_copy(rrows_v.at[b], recv_out.at[pl.ds(off, CHUNK)],
                                 rw_sems[b]),
            )

        idx_cp = {0: idx_load(0)}
        gat_cp = {}
        wb_cp = {}
        for i in range(N_CHUNKS):
            for cp in idx_cp.pop(i):
                cp.wait()
            if i >= 2:
                for cp in wb_cp.pop(i - 2):
                    cp.wait()
            gat_cp[i] = gather(i)
            if i + 1 < N_CHUNKS:
                idx_cp[i + 1] = idx_load(i + 1)
            for cp in gat_cp.pop(i):
                cp.wait()
            wb_cp[i] = writeback(i)
        for i in (N_CHUNKS - 2, N_CHUNKS - 1):
            if i in wb_cp:
                for cp in wb_cp.pop(i):
                    cp.wait()

    return _sc_gather


BE = 2000


def _mlp_body(ea_ref, r_ref, s_ref, w1e_ref, w1r_ref, w1s_ref, b1_ref,
              w2_ref, b2_ref, o_ref):
    bf = jnp.bfloat16
    acc = jnp.dot(r_ref[...], w1r_ref[...], preferred_element_type=jnp.float32)
    acc = acc + jnp.dot(s_ref[...], w1s_ref[...],
                        preferred_element_type=jnp.float32)
    acc = acc + jnp.dot(ea_ref[...], w1e_ref[...],
                        preferred_element_type=jnp.float32)
    h = jnp.maximum(acc + b1_ref[...], 0.0).astype(bf)
    o_ref[...] = (jnp.dot(h, w2_ref[...], preferred_element_type=jnp.float32)
                  + b2_ref[...])


def _full(shape):
    return pl.BlockSpec(shape, lambda i: (0,) * len(shape))


def _mlp(edge_attr, recv_g, send_g, W1e, W1r, W1s, b1, W2, b2):
    n_edges = recv_g.shape[0]
    grid = (n_edges // BE,)
    return pl.pallas_call(
        _mlp_body,
        grid=grid,
        in_specs=[
            pl.BlockSpec((BE, D_EDGE), lambda i: (i, 0)),
            pl.BlockSpec((BE, D_FEAT), lambda i: (i, 0)),
            pl.BlockSpec((BE, D_FEAT), lambda i: (i, 0)),
            _full((D_EDGE, D_HID)),
            _full((D_FEAT, D_HID)),
            _full((D_FEAT, D_HID)),
            _full((1, D_HID)),
            _full((D_HID, D_OUT)),
            _full((1, D_OUT)),
        ],
        out_specs=pl.BlockSpec((BE, D_OUT), lambda i: (i, 0)),
        out_shape=jax.ShapeDtypeStruct((n_edges, D_OUT), jnp.float32),
        compiler_params=pltpu.CompilerParams(
            dimension_semantics=("arbitrary",),
        ),
    )(edge_attr, recv_g, send_g, W1e, W1r, W1s, b1, W2, b2)


def _unpack_bf16(g):
    return lax.bitcast_convert_type(g, jnp.bfloat16).reshape(g.shape[0],
                                                             D_FEAT)


def kernel(nodes, edge_attr, senders, receivers, W1, b1, W2, b2):
    gather = _make_sc_gather()
    bf = jnp.bfloat16
    nodes_packed = lax.bitcast_convert_type(
        nodes.astype(bf).reshape(N_NODES, D_PACK, 2), jnp.int32)
    W1bf = W1.astype(bf)
    W1e = W1bf[:D_EDGE]
    W1r = W1bf[D_EDGE:D_EDGE + D_FEAT]
    W1s = W1bf[D_EDGE + D_FEAT:]
    W2bf = W2.astype(bf)
    ea_bf = edge_attr.astype(bf)
    b1r = b1.reshape(1, -1)
    b2r = b2.reshape(1, -1)

    gathered = []
    for s in range(N_SLICES):
        lo = s * E_SLICE
        send_g, recv_g = gather(nodes_packed, senders[lo:lo + E_SLICE],
                                receivers[lo:lo + E_SLICE])
        gathered.append((send_g, recv_g))
    outs = []
    for s in range(N_SLICES):
        lo = s * E_SLICE
        send_g, recv_g = gathered[s]
        outs.append(_mlp(ea_bf[lo:lo + E_SLICE], _unpack_bf16(recv_g),
                         _unpack_bf16(send_g), W1e, W1r, W1s, b1r, W2bf, b2r))
    return jnp.concatenate(outs, axis=0)


# speedup vs baseline: 2.4543x; 2.4543x over previous
import functools

import jax
import jax.numpy as jnp
from jax import lax
from jax.experimental import pallas as pl
from jax.experimental.pallas import tpu as pltpu
from jax.experimental.pallas import tpu_sc as plsc

N_NODES = 10000
N_EDGES = 160000
D_FEAT = 256
D_EDGE = 16
D_HID = 1024
D_OUT = 256
D_PACK = D_FEAT // 2

NUM_CORES = 2
NUM_SUBCORES = 16
N_WORKERS = NUM_CORES * NUM_SUBCORES
N_SLICES = 5
E_SLICE = N_EDGES // N_SLICES
EDGES_PER_WORKER = E_SLICE // N_WORKERS
CHUNK = 40
N_CHUNKS = EDGES_PER_WORKER // CHUNK


@functools.lru_cache(maxsize=None)
def _make_sc_gather():
    mesh = plsc.VectorSubcoreMesh(
        core_axis_name="c", subcore_axis_name="s",
        num_cores=NUM_CORES, num_subcores=NUM_SUBCORES)

    @functools.partial(
        pl.kernel,
        out_type=(
            jax.ShapeDtypeStruct((E_SLICE, D_PACK), jnp.int32),
            jax.ShapeDtypeStruct((E_SLICE, D_PACK), jnp.int32),
        ),
        mesh=mesh,
        scratch_types=[
            pltpu.VMEM((2, CHUNK), jnp.int32),
            pltpu.VMEM((2, CHUNK), jnp.int32),
            pltpu.VMEM((2, CHUNK, D_PACK), jnp.int32),
            pltpu.VMEM((2, CHUNK, D_PACK), jnp.int32),
        ] + [pltpu.SemaphoreType.DMA] * 12,
    )
    def _sc_gather(nodes_hbm, senders_hbm, receivers_hbm,
                   send_out, recv_out, sidx_v, ridx_v, srows_v, rrows_v,
                   *sems):
        (si_sem0, si_sem1, ri_sem0, ri_sem1, sg_sem0, sg_sem1,
         rg_sem0, rg_sem1, sw_sem0, sw_sem1, rw_sem0, rw_sem1) = sems
        si_sems = (si_sem0, si_sem1)
        ri_sems = (ri_sem0, ri_sem1)
        sg_sems = (sg_sem0, sg_sem1)
        rg_sems = (rg_sem0, rg_sem1)
        sw_sems = (sw_sem0, sw_sem1)
        rw_sems = (rw_sem0, rw_sem1)

        wid = lax.axis_index("s") * NUM_CORES + lax.axis_index("c")
        base = wid * EDGES_PER_WORKER

        def idx_load(i):
            b = i % 2
            off = base + i * CHUNK
            return (
                pltpu.async_copy(senders_hbm.at[pl.ds(off, CHUNK)],
                                 sidx_v.at[b], si_sems[b]),
                pltpu.async_copy(receivers_hbm.at[pl.ds(off, CHUNK)],
                                 ridx_v.at[b], ri_sems[b]),
            )

        def gather(i):
            b = i % 2
            return (
                pltpu.async_copy(nodes_hbm.at[sidx_v.at[b]], srows_v.at[b],
                                 sg_sems[b]),
                pltpu.async_copy(nodes_hbm.at[ridx_v.at[b]], rrows_v.at[b],
                                 rg_sems[b]),
            )

        def writeback(i):
            b = i % 2
            off = base + i * CHUNK
            return (
                pltpu.async_copy(srows_v.at[b], send_out.at[pl.ds(off, CHUNK)],
                                 sw_sems[b]),
                pltpu.async_copy(rrows_v.at[b], recv_out.at[pl.ds(off, CHUNK)],
                                 rw_sems[b]),
            )

        idx_cp = {0: idx_load(0)}
        gat_cp = {}
        wb_cp = {}
        for i in range(N_CHUNKS):
            for cp in idx_cp.pop(i):
                cp.wait()
            if i >= 2:
                for cp in wb_cp.pop(i - 2):
                    cp.wait()
            gat_cp[i] = gather(i)
            if i + 1 < N_CHUNKS:
                idx_cp[i + 1] = idx_load(i + 1)
            for cp in gat_cp.pop(i):
                cp.wait()
            wb_cp[i] = writeback(i)
        for i in (N_CHUNKS - 2, N_CHUNKS - 1):
            if i in wb_cp:
                for cp in wb_cp.pop(i):
                    cp.wait()

    return _sc_gather


BE = 2000


def _unpack_halves(x_i32):
    bf = jnp.bfloat16
    even = lax.bitcast_convert_type(x_i32 << 16, jnp.float32).astype(bf)
    odd = lax.bitcast_convert_type(
        x_i32 & jnp.int32(-65536), jnp.float32).astype(bf)
    return even, odd


def _mlp_body(ea_ref, r_ref, s_ref, w1e_ref, w1r_ev_ref, w1r_od_ref,
              w1s_ev_ref, w1s_od_ref, b1_ref, w2_ref, b2_ref, o_ref):
    bf = jnp.bfloat16
    r_ev, r_od = _unpack_halves(r_ref[...])
    s_ev, s_od = _unpack_halves(s_ref[...])
    acc = jnp.dot(r_ev, w1r_ev_ref[...], preferred_element_type=jnp.float32)
    acc = acc + jnp.dot(r_od, w1r_od_ref[...],
                        preferred_element_type=jnp.float32)
    acc = acc + jnp.dot(s_ev, w1s_ev_ref[...],
                        preferred_element_type=jnp.float32)
    acc = acc + jnp.dot(s_od, w1s_od_ref[...],
                        preferred_element_type=jnp.float32)
    acc = acc + jnp.dot(ea_ref[...], w1e_ref[...],
                        preferred_element_type=jnp.float32)
    h = jnp.maximum(acc + b1_ref[...], 0.0).astype(bf)
    o_ref[...] = (jnp.dot(h, w2_ref[...], preferred_element_type=jnp.float32)
                  + b2_ref[...])


def _full(shape):
    return pl.BlockSpec(shape, lambda i: (0,) * len(shape))


def _mlp(edge_attr, recv_g, send_g, W1e, W1r_ev, W1r_od, W1s_ev, W1s_od,
         b1, W2, b2):
    n_edges = recv_g.shape[0]
    grid = (n_edges // BE,)
    return pl.pallas_call(
        _mlp_body,
        grid=grid,
        in_specs=[
            pl.BlockSpec((BE, D_EDGE), lambda i: (i, 0)),
            pl.BlockSpec((BE, D_PACK), lambda i: (i, 0)),
            pl.BlockSpec((BE, D_PACK), lambda i: (i, 0)),
            _full((D_EDGE, D_HID)),
            _full((D_PACK, D_HID)),
            _full((D_PACK, D_HID)),
            _full((D_PACK, D_HID)),
            _full((D_PACK, D_HID)),
            _full((1, D_HID)),
            _full((D_HID, D_OUT)),
            _full((1, D_OUT)),
        ],
        out_specs=pl.BlockSpec((BE, D_OUT), lambda i: (i, 0)),
        out_shape=jax.ShapeDtypeStruct((n_edges, D_OUT), jnp.float32),
        compiler_params=pltpu.CompilerParams(
            dimension_semantics=("arbitrary",),
        ),
    )(edge_attr, recv_g, send_g, W1e, W1r_ev, W1r_od, W1s_ev, W1s_od,
      b1, W2, b2)


def kernel(nodes, edge_attr, senders, receivers, W1, b1, W2, b2):
    gather = _make_sc_gather()
    bf = jnp.bfloat16
    nodes_packed = lax.bitcast_convert_type(
        nodes.astype(bf).reshape(N_NODES, D_PACK, 2), jnp.int32)
    W1bf = W1.astype(bf)
    W1e = W1bf[:D_EDGE]
    W1r = W1bf[D_EDGE:D_EDGE + D_FEAT]
    W1s = W1bf[D_EDGE + D_FEAT:]
    W1r_ev, W1r_od = W1r[0::2], W1r[1::2]
    W1s_ev, W1s_od = W1s[0::2], W1s[1::2]
    W2bf = W2.astype(bf)
    ea_bf = edge_attr.astype(bf)
    b1r = b1.reshape(1, -1)
    b2r = b2.reshape(1, -1)

    gathered = []
    for s in range(N_SLICES):
        lo = s * E_SLICE
        send_g, recv_g = gather(nodes_packed, senders[lo:lo + E_SLICE],
                                receivers[lo:lo + E_SLICE])
        gathered.append((send_g, recv_g))
    outs = []
    for s in range(N_SLICES):
        lo = s * E_SLICE
        send_g, recv_g = gathered[s]
        outs.append(_mlp(ea_bf[lo:lo + E_SLICE], recv_g, send_g,
                         W1e, W1r_ev, W1r_od, W1s_ev, W1s_od,
                         b1r, W2bf, b2r))
    return jnp.concatenate(outs, axis=0)


# speedup vs baseline: 3.4688x; 1.4133x over previous
import functools

import jax
import jax.numpy as jnp
from jax import lax
from jax.experimental import pallas as pl
from jax.experimental.pallas import tpu as pltpu
from jax.experimental.pallas import tpu_sc as plsc

N_NODES = 10000
N_EDGES = 160000
D_FEAT = 256
D_EDGE = 16
D_HID = 1024
D_OUT = 256
D_PACK = D_FEAT // 2

NUM_CORES = 2
NUM_SUBCORES = 16
N_WORKERS = NUM_CORES * NUM_SUBCORES
N_SLICES = 5
E_SLICE = N_EDGES // N_SLICES
EDGES_PER_WORKER = E_SLICE // N_WORKERS
CHUNK = 40
N_CHUNKS = EDGES_PER_WORKER // CHUNK


@functools.lru_cache(maxsize=None)
def _make_sc_gather():
    mesh = plsc.VectorSubcoreMesh(
        core_axis_name="c", subcore_axis_name="s",
        num_cores=NUM_CORES, num_subcores=NUM_SUBCORES)

    @functools.partial(
        pl.kernel,
        out_type=(
            jax.ShapeDtypeStruct((E_SLICE, D_PACK), jnp.int32),
            jax.ShapeDtypeStruct((E_SLICE, D_PACK), jnp.int32),
        ),
        mesh=mesh,
        scratch_types=[
            pltpu.VMEM((2, CHUNK), jnp.int32),
            pltpu.VMEM((2, CHUNK), jnp.int32),
            pltpu.VMEM((2, CHUNK, D_PACK), jnp.int32),
            pltpu.VMEM((2, CHUNK, D_PACK), jnp.int32),
        ] + [pltpu.SemaphoreType.DMA] * 12,
    )
    def _sc_gather(nodes_hbm, senders_hbm, receivers_hbm,
                   send_out, recv_out, sidx_v, ridx_v, srows_v, rrows_v,
                   *sems):
        (si_sem0, si_sem1, ri_sem0, ri_sem1, sg_sem0, sg_sem1,
         rg_sem0, rg_sem1, sw_sem0, sw_sem1, rw_sem0, rw_sem1) = sems
        si_sems = (si_sem0, si_sem1)
        ri_sems = (ri_sem0, ri_sem1)
        sg_sems = (sg_sem0, sg_sem1)
        rg_sems = (rg_sem0, rg_sem1)
        sw_sems = (sw_sem0, sw_sem1)
        rw_sems = (rw_sem0, rw_sem1)

        wid = lax.axis_index("s") * NUM_CORES + lax.axis_index("c")
        base = wid * EDGES_PER_WORKER

        def idx_load(i):
            b = i % 2
            off = base + i * CHUNK
            return (
                pltpu.async_copy(senders_hbm.at[pl.ds(off, CHUNK)],
                                 sidx_v.at[b], si_sems[b]),
                pltpu.async_copy(receivers_hbm.at[pl.ds(off, CHUNK)],
                                 ridx_v.at[b], ri_sems[b]),
            )

        def gather(i):
            b = i % 2
            return (
                pltpu.async_copy(nodes_hbm.at[sidx_v.at[b]], srows_v.at[b],
                                 sg_sems[b]),
                pltpu.async_copy(nodes_hbm.at[ridx_v.at[b]], rrows_v.at[b],
                                 rg_sems[b]),
            )

        def writeback(i):
            b = i % 2
            off = base + i * CHUNK
            return (
                pltpu.async_copy(srows_v.at[b], send_out.at[pl.ds(off, CHUNK)],
                                 sw_sems[b]),
                pltpu.async_copy(rrows_v.at[b], recv_out.at[pl.ds(off, CHUNK)],
                                 rw_sems[b]),
            )

        idx_cp = {0: idx_load(0)}
        gat_cp = {}
        wb_cp = {}
        for i in range(N_CHUNKS):
            for cp in idx_cp.pop(i):
                cp.wait()
            if i >= 2:
                for cp in wb_cp.pop(i - 2):
                    cp.wait()
            gat_cp[i] = gather(i)
            if i + 1 < N_CHUNKS:
                idx_cp[i + 1] = idx_load(i + 1)
            for cp in gat_cp.pop(i):
                cp.wait()
            wb_cp[i] = writeback(i)
        for i in (N_CHUNKS - 2, N_CHUNKS - 1):
            if i in wb_cp:
                for cp in wb_cp.pop(i):
                    cp.wait()

    return _sc_gather


BE = 2000


def _unpack_halves(x_i32):
    bf = jnp.bfloat16
    even = lax.bitcast_convert_type(x_i32 << 16, jnp.float32).astype(bf)
    odd = lax.bitcast_convert_type(
        x_i32 & jnp.int32(-65536), jnp.float32).astype(bf)
    return even, odd


def _mlp_body(ea_ref, r_ref, s_ref, w1e_ref, w1r_ref, w1s_ref,
              b1_ref, w2_ref, b2_ref, alias_ref, o_ref):
    bf = jnp.bfloat16
    r_ev, r_od = _unpack_halves(r_ref[...])
    s_ev, s_od = _unpack_halves(s_ref[...])
    r = jnp.concatenate([r_ev, r_od], axis=1)
    s = jnp.concatenate([s_ev, s_od], axis=1)
    acc = jnp.dot(r, w1r_ref[...], preferred_element_type=jnp.float32)
    acc = acc + jnp.dot(s, w1s_ref[...], preferred_element_type=jnp.float32)
    acc = acc + jnp.dot(ea_ref[...], w1e_ref[...],
                        preferred_element_type=jnp.float32)
    h = jnp.maximum(acc + b1_ref[...], 0.0).astype(bf)
    o_ref[...] = (jnp.dot(h, w2_ref[...], preferred_element_type=jnp.float32)
                  + b2_ref[...])


def _full(shape):
    return pl.BlockSpec(shape, lambda i: (0,) * len(shape))


def _mlp_body_noalias(ea_ref, r_ref, s_ref, w1e_ref, w1r_ref, w1s_ref,
                      b1_ref, w2_ref, b2_ref, o_ref):
    _mlp_body(ea_ref, r_ref, s_ref, w1e_ref, w1r_ref, w1s_ref,
              b1_ref, w2_ref, b2_ref, None, o_ref)


def _mlp_slice(slice_idx, edge_attr, recv_g, send_g, W1e, W1r, W1s,
               b1, W2, b2, out_prev):
    grid = (E_SLICE // BE,)
    blk0 = slice_idx * (E_SLICE // BE)
    in_specs = [
        pl.BlockSpec((BE, D_EDGE), lambda i: (i, 0)),
        pl.BlockSpec((BE, D_PACK), lambda i: (i, 0)),
        pl.BlockSpec((BE, D_PACK), lambda i: (i, 0)),
        _full((D_EDGE, D_HID)),
        _full((D_FEAT, D_HID)),
        _full((D_FEAT, D_HID)),
        _full((1, D_HID)),
        _full((D_HID, D_OUT)),
        _full((1, D_OUT)),
    ]
    args = [edge_attr, recv_g, send_g, W1e, W1r, W1s, b1, W2, b2]
    if out_prev is None:
        body, aliases = _mlp_body_noalias, {}
    else:
        body, aliases = _mlp_body, {9: 0}
        in_specs.append(pl.BlockSpec(memory_space=pl.ANY))
        args.append(out_prev)
    return pl.pallas_call(
        body,
        grid=grid,
        in_specs=in_specs,
        out_specs=pl.BlockSpec((BE, D_OUT), lambda i, blk0=blk0: (i + blk0, 0)),
        out_shape=jax.ShapeDtypeStruct((N_EDGES, D_OUT), jnp.float32),
        input_output_aliases=aliases,
        compiler_params=pltpu.CompilerParams(
            dimension_semantics=("arbitrary",),
        ),
    )(*args)


def kernel(nodes, edge_attr, senders, receivers, W1, b1, W2, b2):
    gather = _make_sc_gather()
    bf = jnp.bfloat16
    nodes_packed = lax.bitcast_convert_type(
        nodes.astype(bf).reshape(N_NODES, D_PACK, 2), jnp.int32)
    W1bf = W1.astype(bf)
    W1e = W1bf[:D_EDGE]
    W1r = W1bf[D_EDGE:D_EDGE + D_FEAT]
    W1s = W1bf[D_EDGE + D_FEAT:]
    W1r_p = jnp.concatenate([W1r[0::2], W1r[1::2]], axis=0)
    W1s_p = jnp.concatenate([W1s[0::2], W1s[1::2]], axis=0)
    W2bf = W2.astype(bf)
    ea_bf = edge_attr.astype(bf)
    b1r = b1.reshape(1, -1)
    b2r = b2.reshape(1, -1)

    gathered = []
    for s in range(N_SLICES):
        lo = s * E_SLICE
        send_g, recv_g = gather(nodes_packed, senders[lo:lo + E_SLICE],
                                receivers[lo:lo + E_SLICE])
        gathered.append((send_g, recv_g))
    out = None
    for s in range(N_SLICES):
        lo = s * E_SLICE
        send_g, recv_g = gathered[s]
        out = _mlp_slice(s, ea_bf[lo:lo + E_SLICE], recv_g, send_g,
                         W1e, W1r_p, W1s_p, b1r, W2bf, b2r, out)
    return out


# speedup vs baseline: 4.0911x; 1.1794x over previous
import functools

import jax
import jax.numpy as jnp
from jax import lax
from jax.experimental import pallas as pl
from jax.experimental.pallas import tpu as pltpu
from jax.experimental.pallas import tpu_sc as plsc

N_NODES = 10000
N_EDGES = 160000
D_FEAT = 256
D_EDGE = 16
D_HID = 1024
D_OUT = 256
D_PACK = D_FEAT // 2

NUM_CORES = 2
NUM_SUBCORES = 16
N_WORKERS = NUM_CORES * NUM_SUBCORES
N_SLICES = 5
E_SLICE = N_EDGES // N_SLICES
EDGES_PER_WORKER = E_SLICE // N_WORKERS
CHUNK = 40
N_CHUNKS = EDGES_PER_WORKER // CHUNK


@functools.lru_cache(maxsize=None)
def _make_sc_gather():
    mesh = plsc.VectorSubcoreMesh(
        core_axis_name="c", subcore_axis_name="s",
        num_cores=NUM_CORES, num_subcores=NUM_SUBCORES)

    @functools.partial(
        pl.kernel,
        out_type=(
            jax.ShapeDtypeStruct((E_SLICE, D_PACK), jnp.int32),
            jax.ShapeDtypeStruct((E_SLICE, D_PACK), jnp.int32),
        ),
        mesh=mesh,
        scratch_types=[
            pltpu.VMEM((2, CHUNK), jnp.int32),
            pltpu.VMEM((2, CHUNK), jnp.int32),
            pltpu.VMEM((2, CHUNK, D_PACK), jnp.int32),
            pltpu.VMEM((2, CHUNK, D_PACK), jnp.int32),
        ] + [pltpu.SemaphoreType.DMA] * 12,
    )
    def _sc_gather(nodes_hbm, senders_hbm, receivers_hbm,
                   send_out, recv_out, sidx_v, ridx_v, srows_v, rrows_v,
                   *sems):
        (si_sem0, si_sem1, ri_sem0, ri_sem1, sg_sem0, sg_sem1,
         rg_sem0, rg_sem1, sw_sem0, sw_sem1, rw_sem0, rw_sem1) = sems
        si_sems = (si_sem0, si_sem1)
        ri_sems = (ri_sem0, ri_sem1)
        sg_sems = (sg_sem0, sg_sem1)
        rg_sems = (rg_sem0, rg_sem1)
        sw_sems = (sw_sem0, sw_sem1)
        rw_sems = (rw_sem0, rw_sem1)

        wid = lax.axis_index("s") * NUM_CORES + lax.axis_index("c")
        base = wid * EDGES_PER_WORKER

        def idx_load(i):
            b = i % 2
            off = base + i * CHUNK
            return (
                pltpu.async_copy(senders_hbm.at[pl.ds(off, CHUNK)],
                                 sidx_v.at[b], si_sems[b]),
                pltpu.async_copy(receivers_hbm.at[pl.ds(off, CHUNK)],
                                 ridx_v.at[b], ri_sems[b]),
            )

        def gather(i):
            b = i % 2
            return (
                pltpu.async_copy(nodes_hbm.at[sidx_v.at[b]], srows_v.at[b],
                                 sg_sems[b]),
                pltpu.async_copy(nodes_hbm.at[ridx_v.at[b]], rrows_v.at[b],
                                 rg_sems[b]),
            )

        def writeback(i):
            b = i % 2
            off = base + i * CHUNK
            return (
                pltpu.async_copy(srows_v.at[b], send_out.at[pl.ds(off, CHUNK)],
                                 sw_sems[b]),
                pltpu.async_copy(rrows_v.at[b], recv_out.at[pl.ds(off, CHUNK)],
                                 rw_sems[b]),
            )

        idx_cp = {0: idx_load(0)}
        gat_cp = {}
        wb_cp = {}
        for i in range(N_CHUNKS):
            for cp in idx_cp.pop(i):
                cp.wait()
            if i >= 2:
                for cp in wb_cp.pop(i - 2):
                    cp.wait()
            gat_cp[i] = gather(i)
            if i + 1 < N_CHUNKS:
                idx_cp[i + 1] = idx_load(i + 1)
            for cp in gat_cp.pop(i):
                cp.wait()
            wb_cp[i] = writeback(i)
        for i in (N_CHUNKS - 2, N_CHUNKS - 1):
            if i in wb_cp:
                for cp in wb_cp.pop(i):
                    cp.wait()

    return _sc_gather


BE = 2000
NODE_BLOCK = 2000


def _pack_body(x_ref, o_ref):
    f32 = jnp.float32
    bf = jnp.bfloat16
    lo = x_ref[:, :D_PACK].astype(bf).astype(f32)
    hi = x_ref[:, D_PACK:].astype(bf).astype(f32)
    lo_b = lax.shift_right_logical(
        lax.bitcast_convert_type(lo, jnp.uint32), jnp.uint32(16))
    hi_b = lax.bitcast_convert_type(hi, jnp.uint32) & jnp.uint32(0xFFFF0000)
    o_ref[...] = lax.bitcast_convert_type(lo_b | hi_b, jnp.int32)


def _pack_nodes(nodes):
    return pl.pallas_call(
        _pack_body,
        grid=(N_NODES // NODE_BLOCK,),
        in_specs=[pl.BlockSpec((NODE_BLOCK, D_FEAT), lambda i: (i, 0))],
        out_specs=pl.BlockSpec((NODE_BLOCK, D_PACK), lambda i: (i, 0)),
        out_shape=jax.ShapeDtypeStruct((N_NODES, D_PACK), jnp.int32),
        compiler_params=pltpu.CompilerParams(
            dimension_semantics=("arbitrary",),
        ),
    )(nodes)


def _unpack_halves(x_i32):
    bf = jnp.bfloat16
    lo = lax.bitcast_convert_type(x_i32 << 16, jnp.float32).astype(bf)
    hi = lax.bitcast_convert_type(
        x_i32 & jnp.int32(-65536), jnp.float32).astype(bf)
    return lo, hi


def _mlp_body(ea_ref, r_ref, s_ref, w1e_ref, w1r_ref, w1s_ref,
              b1_ref, w2_ref, b2_ref, alias_ref, o_ref):
    bf = jnp.bfloat16
    r_lo, r_hi = _unpack_halves(r_ref[...])
    s_lo, s_hi = _unpack_halves(s_ref[...])
    r = jnp.concatenate([r_lo, r_hi], axis=1)
    s = jnp.concatenate([s_lo, s_hi], axis=1)
    acc = jnp.dot(r, w1r_ref[...], preferred_element_type=jnp.float32)
    acc = acc + jnp.dot(s, w1s_ref[...], preferred_element_type=jnp.float32)
    acc = acc + jnp.dot(ea_ref[...], w1e_ref[...],
                        preferred_element_type=jnp.float32)
    h = jnp.maximum(acc + b1_ref[...], 0.0).astype(bf)
    o_ref[...] = (jnp.dot(h, w2_ref[...], preferred_element_type=jnp.float32)
                  + b2_ref[...])


def _full(shape):
    return pl.BlockSpec(shape, lambda i: (0,) * len(shape))


def _mlp_body_noalias(ea_ref, r_ref, s_ref, w1e_ref, w1r_ref, w1s_ref,
                      b1_ref, w2_ref, b2_ref, o_ref):
    _mlp_body(ea_ref, r_ref, s_ref, w1e_ref, w1r_ref, w1s_ref,
              b1_ref, w2_ref, b2_ref, None, o_ref)


def _mlp_slice(slice_idx, edge_attr, recv_g, send_g, W1e, W1r, W1s,
               b1, W2, b2, out_prev):
    grid = (E_SLICE // BE,)
    blk0 = slice_idx * (E_SLICE // BE)
    in_specs = [
        pl.BlockSpec((BE, D_EDGE), lambda i: (i, 0)),
        pl.BlockSpec((BE, D_PACK), lambda i: (i, 0)),
        pl.BlockSpec((BE, D_PACK), lambda i: (i, 0)),
        _full((D_EDGE, D_HID)),
        _full((D_FEAT, D_HID)),
        _full((D_FEAT, D_HID)),
        _full((1, D_HID)),
        _full((D_HID, D_OUT)),
        _full((1, D_OUT)),
    ]
    args = [edge_attr, recv_g, send_g, W1e, W1r, W1s, b1, W2, b2]
    if out_prev is None:
        body, aliases = _mlp_body_noalias, {}
    else:
        body, aliases = _mlp_body, {9: 0}
        in_specs.append(pl.BlockSpec(memory_space=pl.ANY))
        args.append(out_prev)
    return pl.pallas_call(
        body,
        grid=grid,
        in_specs=in_specs,
        out_specs=pl.BlockSpec((BE, D_OUT), lambda i, blk0=blk0: (i + blk0, 0)),
        out_shape=jax.ShapeDtypeStruct((N_EDGES, D_OUT), jnp.float32),
        input_output_aliases=aliases,
        compiler_params=pltpu.CompilerParams(
            dimension_semantics=("arbitrary",),
        ),
    )(*args)


def kernel(nodes, edge_attr, senders, receivers, W1, b1, W2, b2):
    gather = _make_sc_gather()
    bf = jnp.bfloat16
    nodes_packed = _pack_nodes(nodes)
    W1bf = W1.astype(bf)
    W1e = W1bf[:D_EDGE]
    W1r_p = W1bf[D_EDGE:D_EDGE + D_FEAT]
    W1s_p = W1bf[D_EDGE + D_FEAT:]
    W2bf = W2.astype(bf)
    ea_bf = edge_attr.astype(bf)
    b1r = b1.reshape(1, -1)
    b2r = b2.reshape(1, -1)

    gathered = []
    for s in range(N_SLICES):
        lo = s * E_SLICE
        send_g, recv_g = gather(nodes_packed, senders[lo:lo + E_SLICE],
                                receivers[lo:lo + E_SLICE])
        gathered.append((send_g, recv_g))
    out = None
    for s in range(N_SLICES):
        lo = s * E_SLICE
        send_g, recv_g = gathered[s]
        out = _mlp_slice(s, ea_bf[lo:lo + E_SLICE], recv_g, send_g,
                         W1e, W1r_p, W1s_p, b1r, W2bf, b2r, out)
    return out


# speedup vs baseline: 4.2061x; 1.0281x over previous
import functools

import jax
import jax.numpy as jnp
from jax import lax
from jax.experimental import pallas as pl
from jax.experimental.pallas import tpu as pltpu
from jax.experimental.pallas import tpu_sc as plsc

N_NODES = 10000
N_EDGES = 160000
D_FEAT = 256
D_EDGE = 16
D_HID = 1024
D_OUT = 256
D_PACK = D_FEAT // 2

NUM_CORES = 2
NUM_SUBCORES = 16
N_WORKERS = NUM_CORES * NUM_SUBCORES
N_SLICES = 5
E_SLICE = N_EDGES // N_SLICES
EDGES_PER_WORKER = E_SLICE // N_WORKERS
CHUNK = 40
N_CHUNKS = EDGES_PER_WORKER // CHUNK


@functools.lru_cache(maxsize=None)
def _make_sc_gather():
    mesh = plsc.VectorSubcoreMesh(
        core_axis_name="c", subcore_axis_name="s",
        num_cores=NUM_CORES, num_subcores=NUM_SUBCORES)

    @functools.partial(
        pl.kernel,
        out_type=(
            jax.ShapeDtypeStruct((E_SLICE, D_PACK), jnp.int32),
            jax.ShapeDtypeStruct((E_SLICE, D_PACK), jnp.int32),
        ),
        mesh=mesh,
        scratch_types=[
            pltpu.VMEM((2, CHUNK), jnp.int32),
            pltpu.VMEM((2, CHUNK), jnp.int32),
            pltpu.VMEM((2, CHUNK, D_PACK), jnp.int32),
            pltpu.VMEM((2, CHUNK, D_PACK), jnp.int32),
        ] + [pltpu.SemaphoreType.DMA] * 12,
    )
    def _sc_gather(nodes_hbm, senders_hbm, receivers_hbm,
                   send_out, recv_out, sidx_v, ridx_v, srows_v, rrows_v,
                   *sems):
        (si_sem0, si_sem1, ri_sem0, ri_sem1, sg_sem0, sg_sem1,
         rg_sem0, rg_sem1, sw_sem0, sw_sem1, rw_sem0, rw_sem1) = sems
        si_sems = (si_sem0, si_sem1)
        ri_sems = (ri_sem0, ri_sem1)
        sg_sems = (sg_sem0, sg_sem1)
        rg_sems = (rg_sem0, rg_sem1)
        sw_sems = (sw_sem0, sw_sem1)
        rw_sems = (rw_sem0, rw_sem1)

        wid = lax.axis_index("s") * NUM_CORES + lax.axis_index("c")
        base = wid * EDGES_PER_WORKER

        def idx_load(i):
            b = i % 2
            off = base + i * CHUNK
            return (
                pltpu.async_copy(senders_hbm.at[pl.ds(off, CHUNK)],
                                 sidx_v.at[b], si_sems[b]),
                pltpu.async_copy(receivers_hbm.at[pl.ds(off, CHUNK)],
                                 ridx_v.at[b], ri_sems[b]),
            )

        def gather(i):
            b = i % 2
            return (
                pltpu.async_copy(nodes_hbm.at[sidx_v.at[b]], srows_v.at[b],
                                 sg_sems[b]),
                pltpu.async_copy(nodes_hbm.at[ridx_v.at[b]], rrows_v.at[b],
                                 rg_sems[b]),
            )

        def writeback(i):
            b = i % 2
            off = base + i * CHUNK
            return (
                pltpu.async_copy(srows_v.at[b], send_out.at[pl.ds(off, CHUNK)],
                                 sw_sems[b]),
                pltpu.async_copy(rrows_v.at[b], recv_out.at[pl.ds(off, CHUNK)],
                                 rw_sems[b]),
            )

        idx_cp = {0: idx_load(0)}
        gat_cp = {}
        wb_cp = {}
        for i in range(N_CHUNKS):
            for cp in idx_cp.pop(i):
                cp.wait()
            if i >= 2:
                for cp in wb_cp.pop(i - 2):
                    cp.wait()
            gat_cp[i] = gather(i)
            if i + 1 < N_CHUNKS:
                idx_cp[i + 1] = idx_load(i + 1)
            for cp in gat_cp.pop(i):
                cp.wait()
            wb_cp[i] = writeback(i)
        for i in (N_CHUNKS - 2, N_CHUNKS - 1):
            if i in wb_cp:
                for cp in wb_cp.pop(i):
                    cp.wait()

    return _sc_gather


BE = 2000
NODE_BLOCK = 2000


def _pack_body(x_ref, o_ref):
    f32 = jnp.float32
    bf = jnp.bfloat16
    lo = x_ref[:, :D_PACK].astype(bf).astype(f32)
    hi = x_ref[:, D_PACK:].astype(bf).astype(f32)
    lo_b = lax.shift_right_logical(
        lax.bitcast_convert_type(lo, jnp.uint32), jnp.uint32(16))
    hi_b = lax.bitcast_convert_type(hi, jnp.uint32) & jnp.uint32(0xFFFF0000)
    o_ref[...] = lax.bitcast_convert_type(lo_b | hi_b, jnp.int32)


def _pack_nodes(nodes):
    return pl.pallas_call(
        _pack_body,
        grid=(N_NODES // NODE_BLOCK,),
        in_specs=[pl.BlockSpec((NODE_BLOCK, D_FEAT), lambda i: (i, 0))],
        out_specs=pl.BlockSpec((NODE_BLOCK, D_PACK), lambda i: (i, 0)),
        out_shape=jax.ShapeDtypeStruct((N_NODES, D_PACK), jnp.int32),
        compiler_params=pltpu.CompilerParams(
            dimension_semantics=("arbitrary",),
        ),
    )(nodes)


def _unpack_halves(x_i32):
    bf = jnp.bfloat16
    lo = lax.bitcast_convert_type(x_i32 << 16, jnp.float32).astype(bf)
    hi = lax.bitcast_convert_type(
        x_i32 & jnp.int32(-65536), jnp.float32).astype(bf)
    return lo, hi


def _mlp_body(ea_ref, r_ref, s_ref, w1e_ref, w1r_ref, w1s_ref,
              b1_ref, w2_ref, b2_ref, alias_ref, o_ref):
    bf = jnp.bfloat16
    r_lo, r_hi = _unpack_halves(r_ref[...])
    s_lo, s_hi = _unpack_halves(s_ref[...])
    r = jnp.concatenate([r_lo, r_hi], axis=1)
    s = jnp.concatenate([s_lo, s_hi], axis=1)
    acc = jnp.dot(r, w1r_ref[...], preferred_element_type=jnp.float32)
    acc = acc + jnp.dot(s, w1s_ref[...], preferred_element_type=jnp.float32)
    acc = acc + jnp.dot(ea_ref[...].astype(bf), w1e_ref[...],
                        preferred_element_type=jnp.float32)
    h = jnp.maximum(acc + b1_ref[...], 0.0).astype(bf)
    o_ref[...] = (jnp.dot(h, w2_ref[...], preferred_element_type=jnp.float32)
                  + b2_ref[...])


def _full(shape):
    return pl.BlockSpec(shape, lambda i: (0,) * len(shape))


def _mlp_body_noalias(ea_ref, r_ref, s_ref, w1e_ref, w1r_ref, w1s_ref,
                      b1_ref, w2_ref, b2_ref, o_ref):
    _mlp_body(ea_ref, r_ref, s_ref, w1e_ref, w1r_ref, w1s_ref,
              b1_ref, w2_ref, b2_ref, None, o_ref)


def _mlp_slice(slice_idx, edge_attr, recv_g, send_g, W1e, W1r, W1s,
               b1, W2, b2, out_prev):
    grid = (E_SLICE // BE,)
    blk0 = slice_idx * (E_SLICE // BE)
    in_specs = [
        pl.BlockSpec((BE, D_EDGE), lambda i, blk0=blk0: (i + blk0, 0)),
        pl.BlockSpec((BE, D_PACK), lambda i: (i, 0)),
        pl.BlockSpec((BE, D_PACK), lambda i: (i, 0)),
        _full((D_EDGE, D_HID)),
        _full((D_FEAT, D_HID)),
        _full((D_FEAT, D_HID)),
        _full((1, D_HID)),
        _full((D_HID, D_OUT)),
        _full((1, D_OUT)),
    ]
    args = [edge_attr, recv_g, send_g, W1e, W1r, W1s, b1, W2, b2]
    if out_prev is None:
        body, aliases = _mlp_body_noalias, {}
    else:
        body, aliases = _mlp_body, {9: 0}
        in_specs.append(pl.BlockSpec(memory_space=pl.ANY))
        args.append(out_prev)
    return pl.pallas_call(
        body,
        grid=grid,
        in_specs=in_specs,
        out_specs=pl.BlockSpec((BE, D_OUT), lambda i, blk0=blk0: (i + blk0, 0)),
        out_shape=jax.ShapeDtypeStruct((N_EDGES, D_OUT), jnp.float32),
        input_output_aliases=aliases,
        compiler_params=pltpu.CompilerParams(
            dimension_semantics=("arbitrary",),
        ),
    )(*args)


def kernel(nodes, edge_attr, senders, receivers, W1, b1, W2, b2):
    gather = _make_sc_gather()
    bf = jnp.bfloat16
    nodes_packed = _pack_nodes(nodes)
    W1bf = W1.astype(bf)
    W1e = W1bf[:D_EDGE]
    W1r_p = W1bf[D_EDGE:D_EDGE + D_FEAT]
    W1s_p = W1bf[D_EDGE + D_FEAT:]
    W2bf = W2.astype(bf)
    b1r = b1.reshape(1, -1)
    b2r = b2.reshape(1, -1)

    gathered = []
    for s in range(N_SLICES):
        lo = s * E_SLICE
        send_g, recv_g = gather(nodes_packed, senders[lo:lo + E_SLICE],
                                receivers[lo:lo + E_SLICE])
        gathered.append((send_g, recv_g))
    out = None
    for s in range(N_SLICES):
        send_g, recv_g = gathered[s]
        out = _mlp_slice(s, edge_attr, recv_g, send_g,
                         W1e, W1r_p, W1s_p, b1r, W2bf, b2r, out)
    return out


# speedup vs baseline: 4.2524x; 1.0110x over previous
import functools

import jax
import jax.numpy as jnp
from jax import lax
from jax.experimental import pallas as pl
from jax.experimental.pallas import tpu as pltpu
from jax.experimental.pallas import tpu_sc as plsc

N_NODES = 10000
N_EDGES = 160000
D_FEAT = 256
D_EDGE = 16
D_HID = 1024
D_OUT = 256
D_PACK = D_FEAT // 2

NUM_CORES = 2
NUM_SUBCORES = 16
N_WORKERS = NUM_CORES * NUM_SUBCORES
N_SLICES = 5
E_SLICE = N_EDGES // N_SLICES
EDGES_PER_WORKER = E_SLICE // N_WORKERS
CHUNK = 40
N_CHUNKS = EDGES_PER_WORKER // CHUNK


@functools.lru_cache(maxsize=None)
def _make_sc_gather():
    mesh = plsc.VectorSubcoreMesh(
        core_axis_name="c", subcore_axis_name="s",
        num_cores=NUM_CORES, num_subcores=NUM_SUBCORES)

    @functools.partial(
        pl.kernel,
        out_type=(
            jax.ShapeDtypeStruct((E_SLICE, D_PACK), jnp.int32),
            jax.ShapeDtypeStruct((E_SLICE, D_PACK), jnp.int32),
        ),
        mesh=mesh,
        scratch_types=[
            pltpu.VMEM((2, CHUNK), jnp.int32),
            pltpu.VMEM((2, CHUNK), jnp.int32),
            pltpu.VMEM((2, CHUNK, D_PACK), jnp.int32),
            pltpu.VMEM((2, CHUNK, D_PACK), jnp.int32),
        ] + [pltpu.SemaphoreType.DMA] * 12,
    )
    def _sc_gather(nodes_hbm, senders_hbm, receivers_hbm,
                   send_out, recv_out, sidx_v, ridx_v, srows_v, rrows_v,
                   *sems):
        (si_sem0, si_sem1, ri_sem0, ri_sem1, sg_sem0, sg_sem1,
         rg_sem0, rg_sem1, sw_sem0, sw_sem1, rw_sem0, rw_sem1) = sems
        si_sems = (si_sem0, si_sem1)
        ri_sems = (ri_sem0, ri_sem1)
        sg_sems = (sg_sem0, sg_sem1)
        rg_sems = (rg_sem0, rg_sem1)
        sw_sems = (sw_sem0, sw_sem1)
        rw_sems = (rw_sem0, rw_sem1)

        wid = lax.axis_index("s") * NUM_CORES + lax.axis_index("c")
        base = wid * EDGES_PER_WORKER

        def idx_load(i):
            b = i % 2
            off = base + i * CHUNK
            return (
                pltpu.async_copy(senders_hbm.at[pl.ds(off, CHUNK)],
                                 sidx_v.at[b], si_sems[b]),
                pltpu.async_copy(receivers_hbm.at[pl.ds(off, CHUNK)],
                                 ridx_v.at[b], ri_sems[b]),
            )

        def gather(i):
            b = i % 2
            return (
                pltpu.async_copy(nodes_hbm.at[sidx_v.at[b]], srows_v.at[b],
                                 sg_sems[b]),
                pltpu.async_copy(nodes_hbm.at[ridx_v.at[b]], rrows_v.at[b],
                                 rg_sems[b]),
            )

        def writeback(i):
            b = i % 2
            off = base + i * CHUNK
            return (
                pltpu.async_copy(srows_v.at[b], send_out.at[pl.ds(off, CHUNK)],
                                 sw_sems[b]),
                pltpu.async_copy(rrows_v.at[b], recv_out.at[pl.ds(off, CHUNK)],
                                 rw_sems[b]),
            )

        idx_cp = {0: idx_load(0)}
        gat_cp = {}
        wb_cp = {}
        for i in range(N_CHUNKS):
            for cp in idx_cp.pop(i):
                cp.wait()
            if i >= 2:
                for cp in wb_cp.pop(i - 2):
                    cp.wait()
            gat_cp[i] = gather(i)
            if i + 1 < N_CHUNKS:
                idx_cp[i + 1] = idx_load(i + 1)
            for cp in gat_cp.pop(i):
                cp.wait()
            wb_cp[i] = writeback(i)
        for i in (N_CHUNKS - 2, N_CHUNKS - 1):
            if i in wb_cp:
                for cp in wb_cp.pop(i):
                    cp.wait()

    return _sc_gather


BE = 4000
NODE_BLOCK = 2000


def _pack_body(x_ref, o_ref):
    f32 = jnp.float32
    bf = jnp.bfloat16
    lo = x_ref[:, :D_PACK].astype(bf).astype(f32)
    hi = x_ref[:, D_PACK:].astype(bf).astype(f32)
    lo_b = lax.shift_right_logical(
        lax.bitcast_convert_type(lo, jnp.uint32), jnp.uint32(16))
    hi_b = lax.bitcast_convert_type(hi, jnp.uint32) & jnp.uint32(0xFFFF0000)
    o_ref[...] = lax.bitcast_convert_type(lo_b | hi_b, jnp.int32)


def _pack_nodes(nodes):
    return pl.pallas_call(
        _pack_body,
        grid=(N_NODES // NODE_BLOCK,),
        in_specs=[pl.BlockSpec((NODE_BLOCK, D_FEAT), lambda i: (i, 0))],
        out_specs=pl.BlockSpec((NODE_BLOCK, D_PACK), lambda i: (i, 0)),
        out_shape=jax.ShapeDtypeStruct((N_NODES, D_PACK), jnp.int32),
        compiler_params=pltpu.CompilerParams(
            dimension_semantics=("arbitrary",),
        ),
    )(nodes)


def _unpack_halves(x_i32):
    bf = jnp.bfloat16
    lo = lax.bitcast_convert_type(x_i32 << 16, jnp.float32).astype(bf)
    hi = lax.bitcast_convert_type(
        x_i32 & jnp.int32(-65536), jnp.float32).astype(bf)
    return lo, hi


def _mlp_body(ea_ref, r_ref, s_ref, w1e_ref, w1r_ref, w1s_ref,
              b1_ref, w2_ref, b2_ref, alias_ref, o_ref):
    bf = jnp.bfloat16
    r_lo, r_hi = _unpack_halves(r_ref[...])
    s_lo, s_hi = _unpack_halves(s_ref[...])
    r = jnp.concatenate([r_lo, r_hi], axis=1)
    s = jnp.concatenate([s_lo, s_hi], axis=1)
    acc = jnp.dot(r, w1r_ref[...], preferred_element_type=jnp.float32)
    acc = acc + jnp.dot(s, w1s_ref[...], preferred_element_type=jnp.float32)
    acc = acc + jnp.dot(ea_ref[...].astype(bf), w1e_ref[...],
                        preferred_element_type=jnp.float32)
    h = jnp.maximum(acc + b1_ref[...], 0.0).astype(bf)
    o_ref[...] = (jnp.dot(h, w2_ref[...], preferred_element_type=jnp.float32)
                  + b2_ref[...])


def _full(shape):
    return pl.BlockSpec(shape, lambda i: (0,) * len(shape))


def _mlp_body_noalias(ea_ref, r_ref, s_ref, w1e_ref, w1r_ref, w1s_ref,
                      b1_ref, w2_ref, b2_ref, o_ref):
    _mlp_body(ea_ref, r_ref, s_ref, w1e_ref, w1r_ref, w1s_ref,
              b1_ref, w2_ref, b2_ref, None, o_ref)


def _mlp_slice(slice_idx, edge_attr, recv_g, send_g, W1e, W1r, W1s,
               b1, W2, b2, out_prev):
    grid = (E_SLICE // BE,)
    blk0 = slice_idx * (E_SLICE // BE)
    in_specs = [
        pl.BlockSpec((BE, D_EDGE), lambda i, blk0=blk0: (i + blk0, 0)),
        pl.BlockSpec((BE, D_PACK), lambda i: (i, 0)),
        pl.BlockSpec((BE, D_PACK), lambda i: (i, 0)),
        _full((D_EDGE, D_HID)),
        _full((D_FEAT, D_HID)),
        _full((D_FEAT, D_HID)),
        _full((1, D_HID)),
        _full((D_HID, D_OUT)),
        _full((1, D_OUT)),
    ]
    args = [edge_attr, recv_g, send_g, W1e, W1r, W1s, b1, W2, b2]
    if out_prev is None:
        body, aliases = _mlp_body_noalias, {}
    else:
        body, aliases = _mlp_body, {9: 0}
        in_specs.append(pl.BlockSpec(memory_space=pl.ANY))
        args.append(out_prev)
    return pl.pallas_call(
        body,
        grid=grid,
        in_specs=in_specs,
        out_specs=pl.BlockSpec((BE, D_OUT), lambda i, blk0=blk0: (i + blk0, 0)),
        out_shape=jax.ShapeDtypeStruct((N_EDGES, D_OUT), jnp.float32),
        input_output_aliases=aliases,
        compiler_params=pltpu.CompilerParams(
            dimension_semantics=("arbitrary",),
        ),
    )(*args)


def kernel(nodes, edge_attr, senders, receivers, W1, b1, W2, b2):
    gather = _make_sc_gather()
    bf = jnp.bfloat16
    nodes_packed = _pack_nodes(nodes)
    W1bf = W1.astype(bf)
    W1e = W1bf[:D_EDGE]
    W1r_p = W1bf[D_EDGE:D_EDGE + D_FEAT]
    W1s_p = W1bf[D_EDGE + D_FEAT:]
    W2bf = W2.astype(bf)
    b1r = b1.reshape(1, -1)
    b2r = b2.reshape(1, -1)

    gathered = []
    for s in range(N_SLICES):
        lo = s * E_SLICE
        send_g, recv_g = gather(nodes_packed, senders[lo:lo + E_SLICE],
                                receivers[lo:lo + E_SLICE])
        gathered.append((send_g, recv_g))
    out = None
    for s in range(N_SLICES):
        send_g, recv_g = gathered[s]
        out = _mlp_slice(s, edge_attr, recv_g, send_g,
                         W1e, W1r_p, W1s_p, b1r, W2bf, b2r, out)
    return out


# speedup vs baseline: 4.5251x; 1.0641x over previous
import functools

import jax
import jax.numpy as jnp
from jax import lax
from jax.experimental import pallas as pl
from jax.experimental.pallas import tpu as pltpu
from jax.experimental.pallas import tpu_sc as plsc

N_NODES = 10000
N_EDGES = 160000
D_FEAT = 256
D_EDGE = 16
D_HID = 1024
D_OUT = 256
D_PACK = D_FEAT // 2

NUM_CORES = 2
NUM_SUBCORES = 16
N_WORKERS = NUM_CORES * NUM_SUBCORES
N_SLICES = 5
E_SLICE = N_EDGES // N_SLICES
EDGES_PER_WORKER = E_SLICE // N_WORKERS
CHUNK = 40
N_CHUNKS = EDGES_PER_WORKER // CHUNK


@functools.lru_cache(maxsize=None)
def _make_sc_gather():
    mesh = plsc.VectorSubcoreMesh(
        core_axis_name="c", subcore_axis_name="s",
        num_cores=NUM_CORES, num_subcores=NUM_SUBCORES)

    @functools.partial(
        pl.kernel,
        out_type=(
            jax.ShapeDtypeStruct((E_SLICE, D_PACK), jnp.int32),
            jax.ShapeDtypeStruct((E_SLICE, D_PACK), jnp.int32),
        ),
        mesh=mesh,
        scratch_types=[
            pltpu.VMEM((2, CHUNK), jnp.int32),
            pltpu.VMEM((2, CHUNK), jnp.int32),
            pltpu.VMEM((2, CHUNK, D_PACK), jnp.int32),
            pltpu.VMEM((2, CHUNK, D_PACK), jnp.int32),
        ] + [pltpu.SemaphoreType.DMA] * 12,
    )
    def _sc_gather(nodes_hbm, senders_hbm, receivers_hbm,
                   send_out, recv_out, sidx_v, ridx_v, srows_v, rrows_v,
                   *sems):
        (si_sem0, si_sem1, ri_sem0, ri_sem1, sg_sem0, sg_sem1,
         rg_sem0, rg_sem1, sw_sem0, sw_sem1, rw_sem0, rw_sem1) = sems
        si_sems = (si_sem0, si_sem1)
        ri_sems = (ri_sem0, ri_sem1)
        sg_sems = (sg_sem0, sg_sem1)
        rg_sems = (rg_sem0, rg_sem1)
        sw_sems = (sw_sem0, sw_sem1)
        rw_sems = (rw_sem0, rw_sem1)

        wid = lax.axis_index("s") * NUM_CORES + lax.axis_index("c")
        base = wid * EDGES_PER_WORKER

        def idx_load(i):
            b = i % 2
            off = base + i * CHUNK
            return (
                pltpu.async_copy(senders_hbm.at[pl.ds(off, CHUNK)],
                                 sidx_v.at[b], si_sems[b]),
                pltpu.async_copy(receivers_hbm.at[pl.ds(off, CHUNK)],
                                 ridx_v.at[b], ri_sems[b]),
            )

        def gather(i):
            b = i % 2
            return (
                pltpu.async_copy(nodes_hbm.at[sidx_v.at[b]], srows_v.at[b],
                                 sg_sems[b]),
                pltpu.async_copy(nodes_hbm.at[ridx_v.at[b]], rrows_v.at[b],
                                 rg_sems[b]),
            )

        def writeback(i):
            b = i % 2
            off = base + i * CHUNK
            return (
                pltpu.async_copy(srows_v.at[b], send_out.at[pl.ds(off, CHUNK)],
                                 sw_sems[b]),
                pltpu.async_copy(rrows_v.at[b], recv_out.at[pl.ds(off, CHUNK)],
                                 rw_sems[b]),
            )

        idx_cp = {0: idx_load(0)}
        gat_cp = {}
        wb_cp = {}
        for i in range(N_CHUNKS):
            for cp in idx_cp.pop(i):
                cp.wait()
            if i >= 2:
                for cp in wb_cp.pop(i - 2):
                    cp.wait()
            gat_cp[i] = gather(i)
            if i + 1 < N_CHUNKS:
                idx_cp[i + 1] = idx_load(i + 1)
            for cp in gat_cp.pop(i):
                cp.wait()
            wb_cp[i] = writeback(i)
        for i in (N_CHUNKS - 2, N_CHUNKS - 1):
            if i in wb_cp:
                for cp in wb_cp.pop(i):
                    cp.wait()

    return _sc_gather


BE = 3200
NODE_BLOCK = 2000


def _pack_body(x_ref, o_ref):
    f32 = jnp.float32
    bf = jnp.bfloat16
    lo = x_ref[:, :D_PACK].astype(bf).astype(f32)
    hi = x_ref[:, D_PACK:].astype(bf).astype(f32)
    lo_b = lax.shift_right_logical(
        lax.bitcast_convert_type(lo, jnp.uint32), jnp.uint32(16))
    hi_b = lax.bitcast_convert_type(hi, jnp.uint32) & jnp.uint32(0xFFFF0000)
    o_ref[...] = lax.bitcast_convert_type(lo_b | hi_b, jnp.int32)


def _pack_nodes(nodes):
    return pl.pallas_call(
        _pack_body,
        grid=(N_NODES // NODE_BLOCK,),
        in_specs=[pl.BlockSpec((NODE_BLOCK, D_FEAT), lambda i: (i, 0))],
        out_specs=pl.BlockSpec((NODE_BLOCK, D_PACK), lambda i: (i, 0)),
        out_shape=jax.ShapeDtypeStruct((N_NODES, D_PACK), jnp.int32),
        compiler_params=pltpu.CompilerParams(
            dimension_semantics=("arbitrary",),
        ),
    )(nodes)


def _unpack_halves(x_i32):
    bf = jnp.bfloat16
    lo = lax.bitcast_convert_type(x_i32 << 16, jnp.float32).astype(bf)
    hi = lax.bitcast_convert_type(
        x_i32 & jnp.int32(-65536), jnp.float32).astype(bf)
    return lo, hi


def _mlp_body(ea_ref, r_ref, s_ref, w1e_ref, w1r_ref, w1s_ref,
              b1_ref, w2_ref, b2_ref, alias_ref, o_ref):
    bf = jnp.bfloat16
    r_lo, r_hi = _unpack_halves(r_ref[...])
    s_lo, s_hi = _unpack_halves(s_ref[...])
    r = jnp.concatenate([r_lo, r_hi], axis=1)
    s = jnp.concatenate([s_lo, s_hi], axis=1)
    acc = jnp.dot(r, w1r_ref[...], preferred_element_type=jnp.float32)
    acc = acc + jnp.dot(s, w1s_ref[...], preferred_element_type=jnp.float32)
    acc = acc + lax.dot_general(
        ea_ref[...].astype(bf), w1e_ref[...],
        dimension_numbers=(((0,), (0,)), ((), ())),
        preferred_element_type=jnp.float32)
    h = jnp.maximum(acc + b1_ref[...], 0.0).astype(bf)
    o_ref[...] = (jnp.dot(h, w2_ref[...], preferred_element_type=jnp.float32)
                  + b2_ref[...])


def _full(shape):
    return pl.BlockSpec(shape, lambda i: (0,) * len(shape))


def _mlp_body_noalias(ea_ref, r_ref, s_ref, w1e_ref, w1r_ref, w1s_ref,
                      b1_ref, w2_ref, b2_ref, o_ref):
    _mlp_body(ea_ref, r_ref, s_ref, w1e_ref, w1r_ref, w1s_ref,
              b1_ref, w2_ref, b2_ref, None, o_ref)


def _mlp_slice(slice_idx, edge_attr, recv_g, send_g, W1e, W1r, W1s,
               b1, W2, b2, out_prev):
    grid = (E_SLICE // BE,)
    blk0 = slice_idx * (E_SLICE // BE)
    in_specs = [
        pl.BlockSpec((D_EDGE, BE), lambda i, blk0=blk0: (0, i + blk0)),
        pl.BlockSpec((BE, D_PACK), lambda i: (i, 0)),
        pl.BlockSpec((BE, D_PACK), lambda i: (i, 0)),
        _full((D_EDGE, D_HID)),
        _full((D_FEAT, D_HID)),
        _full((D_FEAT, D_HID)),
        _full((1, D_HID)),
        _full((D_HID, D_OUT)),
        _full((1, D_OUT)),
    ]
    args = [edge_attr, recv_g, send_g, W1e, W1r, W1s, b1, W2, b2]
    if out_prev is None:
        body, aliases = _mlp_body_noalias, {}
    else:
        body, aliases = _mlp_body, {9: 0}
        in_specs.append(pl.BlockSpec(memory_space=pl.ANY))
        args.append(out_prev)
    return pl.pallas_call(
        body,
        grid=grid,
        in_specs=in_specs,
        out_specs=pl.BlockSpec((BE, D_OUT), lambda i, blk0=blk0: (i + blk0, 0)),
        out_shape=jax.ShapeDtypeStruct((N_EDGES, D_OUT), jnp.float32),
        input_output_aliases=aliases,
        compiler_params=pltpu.CompilerParams(
            dimension_semantics=("arbitrary",),
        ),
    )(*args)


def kernel(nodes, edge_attr, senders, receivers, W1, b1, W2, b2):
    gather = _make_sc_gather()
    bf = jnp.bfloat16
    nodes_packed = _pack_nodes(nodes)
    W1bf = W1.astype(bf)
    W1e = W1bf[:D_EDGE]
    W1r_p = W1bf[D_EDGE:D_EDGE + D_FEAT]
    W1s_p = W1bf[D_EDGE + D_FEAT:]
    W2bf = W2.astype(bf)
    b1r = b1.reshape(1, -1)
    b2r = b2.reshape(1, -1)

    gathered = []
    for s in range(N_SLICES):
        lo = s * E_SLICE
        send_g, recv_g = gather(nodes_packed, senders[lo:lo + E_SLICE],
                                receivers[lo:lo + E_SLICE])
        gathered.append((send_g, recv_g))
    ea_t = edge_attr.T
    out = None
    for s in range(N_SLICES):
        send_g, recv_g = gathered[s]
        out = _mlp_slice(s, ea_t, recv_g, send_g,
                         W1e, W1r_p, W1s_p, b1r, W2bf, b2r, out)
    return out
